# baseline (speedup 1.0000x reference)
import functools

import jax
import jax.numpy as jnp
from jax import lax
from jax.experimental import pallas as pl
from jax.experimental.pallas import tpu as pltpu
from jax.experimental.pallas import tpu_sc as plsc

_NCHUNK = 4


def _sc_copy(pe2d):
    L, D = pe2d.shape
    info = plsc.get_sparse_core_info()
    nw = info.num_cores * info.num_subcores
    rows_per_w = L // nw
    rows_per_c = rows_per_w // _NCHUNK

    mesh = plsc.VectorSubcoreMesh(core_axis_name="c", subcore_axis_name="s")

    @functools.partial(
        pl.kernel,
        out_type=jax.ShapeDtypeStruct((L, D), pe2d.dtype),
        mesh=mesh,
        scratch_types=(
            [pltpu.VMEM((rows_per_w, D), pe2d.dtype)]
            + [pltpu.SemaphoreType.DMA] * (2 * _NCHUNK)
        ),
    )
    def copy_kernel(pe_hbm, out_hbm, buf, *sems):
        rsems, wsems = sems[:_NCHUNK], sems[_NCHUNK:]
        wid = lax.axis_index("s") * info.num_cores + lax.axis_index("c")
        base = wid * rows_per_w
        reads = []
        for i in range(_NCHUNK):
            reads.append(
                pltpu.make_async_copy(
                    pe_hbm.at[pl.ds(base + i * rows_per_c, rows_per_c)],
                    buf.at[pl.ds(i * rows_per_c, rows_per_c)],
                    rsems[i],
                )
            )
            reads[i].start()
        writes = []
        for i in range(_NCHUNK):
            reads[i].wait()
            writes.append(
                pltpu.make_async_copy(
                    buf.at[pl.ds(i * rows_per_c, rows_per_c)],
                    out_hbm.at[pl.ds(base + i * rows_per_c, rows_per_c)],
                    wsems[i],
                )
            )
            writes[i].start()
        for i in range(_NCHUNK):
            writes[i].wait()

    return copy_kernel(pe2d)


def kernel(x, pe):
    L = x.shape[1]
    pe2d = pe.reshape(pe.shape[1], pe.shape[2])[:L]
    return _sc_copy(pe2d)[None]


# speedup vs baseline: 2.7507x; 2.7507x over previous
import jax
import jax.numpy as jnp
from jax.experimental import pallas as pl


def _copy_body(pe_ref, out_ref):
    out_ref[...] = pe_ref[...]


def kernel(x, pe):
    L = x.shape[1]
    D = pe.shape[2]
    pe2d = pe.reshape(pe.shape[1], D)[:L]
    block = 256
    out = pl.pallas_call(
        _copy_body,
        grid=(L // block,),
        in_specs=[pl.BlockSpec((block, D), lambda i: (i, 0))],
        out_specs=pl.BlockSpec((block, D), lambda i: (i, 0)),
        out_shape=jax.ShapeDtypeStruct((L, D), pe2d.dtype),
    )(pe2d)
    return out[None]
